# baseline (speedup 1.0000x reference)
import functools

import jax
import jax.numpy as jnp
from jax import lax
from jax.experimental import pallas as pl
from jax.experimental.pallas import tpu as pltpu
from jax.experimental.pallas import tpu_sc as plsc

_DIM = 512
_CBG = 32
_CBB = 32
_NG = 100
_NB = 400
_L = 16


@functools.cache
def _make_lookup(B, D, V):
    info = plsc.get_sparse_core_info()
    NC, NS = info.num_cores, info.num_subcores
    NW = NC * NS
    DQ = D // 4
    g_per_w = _NG * _CBG
    b_per_s = _NB * _CBB
    G_TOTAL = NW * g_per_w
    assert G_TOTAL + (NW // 4) * b_per_s == B
    NP = _NG // 2
    assert _NB // 8 == NP
    mesh = plsc.VectorSubcoreMesh(core_axis_name="c", subcore_axis_name="s")

    @functools.partial(
        pl.kernel,
        mesh=mesh,
        out_type=jax.ShapeDtypeStruct((B, D), jnp.float32),
        scratch_types=[
            pltpu.VMEM((g_per_w,), jnp.int32),
            pltpu.VMEM((b_per_s,), jnp.int32),
            pltpu.VMEM((V, DQ), jnp.float32),
            [pltpu.VMEM((_CBG, D), jnp.float32) for _ in range(2)],
            [pltpu.VMEM((_CBB, DQ), jnp.float32) for _ in range(2)],
            [pltpu.SemaphoreType.DMA for _ in range(2)],
            [pltpu.SemaphoreType.DMA for _ in range(2)],
            [pltpu.SemaphoreType.DMA for _ in range(2)],
        ],
    )
    def lookup(table_hbm, idx_hbm, out_hbm,
               idx_vg, idx_vb, tbl_v, gbuf, bbuf, sg, ssg, ssb):
        wid = lax.axis_index("s") * NC + lax.axis_index("c")
        gbase = wid * g_per_w
        span = wid // 4
        quarter = wid % 4
        bbase = G_TOTAL + span * b_per_s
        col = quarter * DQ
        pltpu.sync_copy(idx_hbm.at[pl.ds(gbase, g_per_w)], idx_vg)
        pltpu.sync_copy(idx_hbm.at[pl.ds(bbase, b_per_s)], idx_vb)
        pltpu.sync_copy(table_hbm.at[:, pl.ds(col, DQ)], tbl_v)

        def g_start(i, j):
            pltpu.async_copy(
                table_hbm.at[idx_vg.at[pl.ds(i * _CBG, _CBG)]], gbuf[j],
                sg[j])

        def g_wait(i, j):
            pltpu.make_async_copy(
                table_hbm.at[idx_vg.at[pl.ds(i * _CBG, _CBG)]], gbuf[j],
                sg[j]).wait()

        def g_out(i, j):
            pltpu.async_copy(
                gbuf[j], out_hbm.at[pl.ds(gbase + i * _CBG, _CBG)], ssg[j])

        def g_out_wait(i, j):
            pltpu.make_async_copy(
                gbuf[j], out_hbm.at[pl.ds(gbase + i * _CBG, _CBG)],
                ssg[j]).wait()

        def build(i, j):
            off = i * _CBB
            for g in range(_CBB // _L):
                vec = idx_vb[pl.ds(off + g * _L, _L)]
                rs = [vec[k] for k in range(_L)]

                @plsc.parallel_loop(0, DQ // _L, 1, unroll=DQ // _L)
                def col_body(jj):
                    for k in range(_L):
                        bbuf[j][g * _L + k, pl.ds(jj * _L, _L)] = (
                            tbl_v[rs[k], pl.ds(jj * _L, _L)])

        def b_out(i, j):
            pltpu.async_copy(
                bbuf[j],
                out_hbm.at[pl.ds(bbase + i * _CBB, _CBB), pl.ds(col, DQ)],
                ssb[j])

        def b_out_wait(i, j):
            pltpu.make_async_copy(
                bbuf[j],
                out_hbm.at[pl.ds(bbase + i * _CBB, _CBB), pl.ds(col, DQ)],
                ssb[j]).wait()

        def body(p, carry):
            for t in range(2):
                ig = 2 * p + t
                jg = t

                @pl.when(ig >= 1)
                def _():
                    g_wait(ig - 1, 1 - jg)
                    g_out(ig - 1, 1 - jg)

                @pl.when(ig >= 2)
                def _():
                    g_out_wait(ig - 2, jg)

                g_start(ig, jg)

                for u in range(4):
                    ib = 8 * p + 4 * t + u
                    jb = (4 * t + u) % 2

                    @pl.when(ib >= 2)
                    def _():
                        b_out_wait(ib - 2, jb)

                    build(ib, jb)
                    b_out(ib, jb)
            return carry

        lax.fori_loop(0, NP, body, 0)
        g_wait(_NG - 1, (_NG - 1) % 2)
        g_out(_NG - 1, (_NG - 1) % 2)
        g_out_wait(_NG - 2, (_NG - 2) % 2)
        g_out_wait(_NG - 1, (_NG - 1) % 2)
        b_out_wait(_NB - 2, 0)
        b_out_wait(_NB - 1, 1)

    return lookup


def kernel(ids, kernel):
    rows, cols = ids.shape
    B = rows * cols
    idx = ids.reshape(B).astype(jnp.int32)
    out = _make_lookup(B, _DIM, kernel.shape[0])(kernel, idx)
    return out.reshape(rows, cols, _DIM)
